# baseline (speedup 1.0000x reference)
import functools

import jax
import jax.numpy as jnp
from jax import lax
from jax.experimental import pallas as pl
from jax.experimental.pallas import tpu as pltpu
from jax.experimental.pallas import tpu_sc as plsc

N_NODES = 10000
N_EDGES = 320000
D_FEAT = 128

NC = 2
NS = 16
L = 16
NW = NC * NS
E_PER_W = N_EDGES // NW
C = 80
N_CHUNK = E_PER_W // C
NBUF = 4
UNROLL = 4
NJ = D_FEAT // L
PAD_W = 17


def _sqrt16(y):
    i = lax.bitcast_convert_type(y, jnp.int32)
    i = jnp.int32(0x1FBD1DF5) + lax.shift_right_logical(i, 1)
    g = lax.bitcast_convert_type(i, jnp.float32)
    g = 0.5 * (g + y / g)
    g = 0.5 * (g + y / g)
    g = 0.5 * (g + y / g)
    return g


def _edge_acc(hb, tb, e):
    parts = []
    for j in range(D_FEAT // (2 * L)):
        h2 = plsc.bitcast(hb[e, pl.ds(j * L, L)], jnp.bfloat16)
        t2 = plsc.bitcast(tb[e, pl.ds(j * L, L)], jnp.bfloat16)
        m2 = h2 * t2
        m0, m1 = plsc.unpack(m2, format=plsc.PackFormat.INTERLEAVED)
        parts.append(m0 * m0)
        parts.append(m1 * m1)
    while len(parts) > 1:
        parts = [a + b for a, b in zip(parts[::2], parts[1::2])]
    return parts[0]


def _score_kernel(x_hbm, src_hbm, dst_hbm, out_hbm,
                  sidx_v, didx_v, out_v, head_v, tail_v, pad_v, sems):
    wid = lax.axis_index("s") * NC + lax.axis_index("c")
    base = pl.multiple_of(wid * E_PER_W, 8)

    pltpu.sync_copy(src_hbm.at[pl.ds(base, E_PER_W)], sidx_v)
    pltpu.sync_copy(dst_hbm.at[pl.ds(base, E_PER_W)], didx_v)

    col_base = lax.iota(jnp.int32, L) * PAD_W

    def start_gathers(ci, b):
        off = pl.multiple_of(ci * C, 8)
        pltpu.async_copy(x_hbm.at[sidx_v.at[pl.ds(off, C)]], head_v.at[b],
                         sems.at[b])
        pltpu.async_copy(x_hbm.at[didx_v.at[pl.ds(off, C)]], tail_v.at[b],
                         sems.at[b])

    def drain(b):
        pltpu.make_async_copy(x_hbm.at[pl.ds(0, C)], head_v.at[b],
                              sems.at[b]).wait()
        pltpu.make_async_copy(x_hbm.at[pl.ds(0, C)], tail_v.at[b],
                              sems.at[b]).wait()

    def compute(ci, b):
        hb = head_v.at[b]
        tb = tail_v.at[b]
        obase = ci * C

        @plsc.parallel_loop(0, C // L)
        def _(g):
            gbase = g * L
            for e in range(L):
                pad_v[pl.ds((gbase + e) * PAD_W, L)] = \
                    _edge_acc(hb, tb, gbase + e)
            gcol = col_base + g * (L * PAD_W)
            cols = [plsc.load_gather(pad_v, [gcol + j]) for j in range(L)]
            while len(cols) > 1:
                cols = [a + b for a, b in zip(cols[::2], cols[1::2])]
            out_v[pl.ds(obase + gbase, L)] = _sqrt16(cols[0])

    for b in range(NBUF - 1):
        start_gathers(b, b)

    def chunk_body(i, carry):
        b = lax.rem(i, NBUF)
        drain(b)

        @pl.when(i < N_CHUNK - (NBUF - 1))
        def _():
            start_gathers(i + NBUF - 1, lax.rem(i + NBUF - 1, NBUF))

        compute(i, b)
        return carry

    lax.fori_loop(0, N_CHUNK, chunk_body, 0)

    pltpu.sync_copy(out_v, out_hbm.at[pl.ds(base, E_PER_W)])


@jax.jit
def kernel(x, edge_index):
    xh = lax.bitcast_convert_type(
        x.astype(jnp.bfloat16).reshape(N_NODES, D_FEAT // 2, 2), jnp.int32)
    src = edge_index[0]
    dst = edge_index[1]
    mesh = plsc.VectorSubcoreMesh(
        core_axis_name="c", subcore_axis_name="s", num_cores=NC, num_subcores=NS)
    f = functools.partial(
        pl.kernel,
        out_type=jax.ShapeDtypeStruct((N_EDGES,), jnp.float32),
        mesh=mesh,
        scratch_types=[
            pltpu.VMEM((E_PER_W,), jnp.int32),
            pltpu.VMEM((E_PER_W,), jnp.int32),
            pltpu.VMEM((E_PER_W,), jnp.float32),
            pltpu.VMEM((NBUF, C, D_FEAT // 2), jnp.int32),
            pltpu.VMEM((NBUF, C, D_FEAT // 2), jnp.int32),
            pltpu.VMEM((C * PAD_W,), jnp.float32),
            pltpu.SemaphoreType.DMA((NBUF,)),
        ],
        compiler_params=pltpu.CompilerParams(needs_layout_passes=False, use_tc_tiling_on_sc=False),
    )(_score_kernel)
    return f(xh, src, dst)


# speedup vs baseline: 1.1711x; 1.1711x over previous
import functools

import jax
import jax.numpy as jnp
from jax import lax
from jax.experimental import pallas as pl
from jax.experimental.pallas import tpu as pltpu
from jax.experimental.pallas import tpu_sc as plsc

N_NODES = 10000
N_EDGES = 320000
D_FEAT = 128

NC = 2
NS = 16
L = 16
NW = NC * NS
E_PER_W = N_EDGES // NW
C = 80
N_CHUNK = E_PER_W // C
NBUF = 4
UNROLL = 4
NJ = D_FEAT // L
PAD_W = 17


def _sqrt16(y):
    i = lax.bitcast_convert_type(y, jnp.int32)
    i = jnp.int32(0x1FBD1DF5) + lax.shift_right_logical(i, 1)
    g = lax.bitcast_convert_type(i, jnp.float32)
    g = 0.5 * (g + y / g)
    g = 0.5 * (g + y / g)
    g = 0.5 * (g + y / g)
    return g


def _edge_acc(hb, tb, e):
    parts = []
    for j in range(D_FEAT // (2 * L)):
        h2 = plsc.bitcast(hb[e, pl.ds(j * L, L)], jnp.bfloat16)
        t2 = plsc.bitcast(tb[e, pl.ds(j * L, L)], jnp.bfloat16)
        m2 = h2 * t2
        m0, m1 = plsc.unpack(m2, format=plsc.PackFormat.INTERLEAVED)
        parts.append(m0 * m0)
        parts.append(m1 * m1)
    while len(parts) > 1:
        parts = [a + b for a, b in zip(parts[::2], parts[1::2])]
    return parts[0]


def _score_kernel(x_hbm, src_hbm, dst_hbm, out_hbm,
                  sidx_v, didx_v, out_v, head_v, tail_v, pad_v, sems):
    wid = lax.axis_index("s") * NC + lax.axis_index("c")
    base = pl.multiple_of(wid * E_PER_W, 8)

    pltpu.sync_copy(src_hbm.at[pl.ds(base, E_PER_W)], sidx_v)
    pltpu.sync_copy(dst_hbm.at[pl.ds(base, E_PER_W)], didx_v)

    col_base = lax.iota(jnp.int32, L) * PAD_W

    def start_gathers(ci, b):
        off = pl.multiple_of(ci * C, 8)
        pltpu.async_copy(x_hbm.at[sidx_v.at[pl.ds(off, C)]], head_v.at[b],
                         sems.at[b])
        pltpu.async_copy(x_hbm.at[didx_v.at[pl.ds(off, C)]], tail_v.at[b],
                         sems.at[b])

    def drain(b):
        pltpu.make_async_copy(x_hbm.at[pl.ds(0, C)], head_v.at[b],
                              sems.at[b]).wait()
        pltpu.make_async_copy(x_hbm.at[pl.ds(0, C)], tail_v.at[b],
                              sems.at[b]).wait()

    def compute(ci, b):
        hb = head_v.at[b]
        tb = tail_v.at[b]
        obase = ci * C

        @plsc.parallel_loop(0, C, unroll=UNROLL)
        def _(e):
            pad_v[pl.ds(e * PAD_W, L)] = _edge_acc(hb, tb, e)

        @plsc.parallel_loop(0, C // L)
        def _(g):
            gcol = col_base + g * (L * PAD_W)
            cols = [plsc.load_gather(pad_v, [gcol + j]) for j in range(L)]
            while len(cols) > 1:
                cols = [a + b for a, b in zip(cols[::2], cols[1::2])]
            out_v[pl.ds(obase + g * L, L)] = _sqrt16(cols[0])

    for b in range(NBUF - 1):
        start_gathers(b, b)

    def chunk_body(i, carry):
        b = lax.rem(i, NBUF)
        drain(b)

        @pl.when(i < N_CHUNK - (NBUF - 1))
        def _():
            start_gathers(i + NBUF - 1, lax.rem(i + NBUF - 1, NBUF))

        compute(i, b)
        return carry

    lax.fori_loop(0, N_CHUNK, chunk_body, 0)

    pltpu.sync_copy(out_v, out_hbm.at[pl.ds(base, E_PER_W)])


@jax.jit
def kernel(x, edge_index):
    xh = lax.bitcast_convert_type(
        x.astype(jnp.bfloat16).reshape(N_NODES, D_FEAT // 2, 2), jnp.int32)
    src = edge_index[0]
    dst = edge_index[1]
    mesh = plsc.VectorSubcoreMesh(
        core_axis_name="c", subcore_axis_name="s", num_cores=NC, num_subcores=NS)
    f = functools.partial(
        pl.kernel,
        out_type=jax.ShapeDtypeStruct((N_EDGES,), jnp.float32),
        mesh=mesh,
        scratch_types=[
            pltpu.VMEM((E_PER_W,), jnp.int32),
            pltpu.VMEM((E_PER_W,), jnp.int32),
            pltpu.VMEM((E_PER_W,), jnp.float32),
            pltpu.VMEM((NBUF, C, D_FEAT // 2), jnp.int32),
            pltpu.VMEM((NBUF, C, D_FEAT // 2), jnp.int32),
            pltpu.VMEM((C * PAD_W,), jnp.float32),
            pltpu.SemaphoreType.DMA((NBUF,)),
        ],
        compiler_params=pltpu.CompilerParams(needs_layout_passes=False, use_tc_tiling_on_sc=False),
    )(_score_kernel)
    return f(xh, src, dst)
